# Initial kernel scaffold; baseline (speedup 1.0000x reference)
import jax
import jax.numpy as jnp
from jax.experimental import pallas as pl


def kernel(g, h, r, norm, embedding, W_rel, loop_weight, h_bias, ln_gamma, ln_beta):
    raise NotImplementedError("write your pallas kernel here")


# speedup vs baseline: 25.2914x; 25.2914x over previous
import jax
import jax.numpy as jnp
from jax import lax
from jax.experimental import pallas as pl
from jax.experimental.pallas import tpu as pltpu
from jax.experimental.pallas import tpu_sc as plsc

H = 128
NB = 4
BLK = H // NB
L = 16
NC = 2
NS = 16
NW = NC * NS
C = 80


def _matmul_body(x_ref, w_ref, y_ref):
    y_ref[0] = jnp.dot(x_ref[...], w_ref[0], preferred_element_type=jnp.float32)


def _epilogue_body(p_ref, ys_ref, prm_ref, o_ref):
    agg = p_ref[0] + p_ref[1]
    mu = jnp.mean(agg, axis=-1, keepdims=True)
    var = jnp.mean((agg - mu) ** 2, axis=-1, keepdims=True)
    ln = (agg - mu) * lax.rsqrt(var + 1e-5)
    o_ref[...] = (ln * prm_ref[0, :] + prm_ref[1, :] + prm_ref[2, :]
                  + ys_ref[...])


def _make_sc_body(n_nodes, n_pad, n_edges):
    ept = n_edges // NW
    n_chunks = ept // C
    rpt = n_pad // NS
    assert ept % C == 0 and n_pad % (NS * C) == 0 and rpt % 8 == 0

    def body(y_ref, src_ref, dst_ref, r_ref, norm_ref, out_ref,
             src_v, dst_v, r_v, norm_v, gidx, rows, sh_agg, sem):
        c = lax.axis_index("c")
        s = lax.axis_index("s")
        wid = s * NC + c

        zero16 = jnp.zeros((L,), jnp.float32)

        def zrow(i, carry):
            for j in range(H // L):
                rows[i, pl.ds(j * L, L)] = zero16
            return carry

        lax.fori_loop(0, C, zrow, 0)
        for k in range(rpt // C):
            pltpu.sync_copy(rows, sh_agg.at[pl.ds(s * rpt + k * C, C)])
        plsc.subcore_barrier()

        def chunk(i, carry):
            base = pl.multiple_of(wid * ept + i * C, 8)
            pltpu.sync_copy(src_ref.at[pl.ds(base, C)], src_v)
            pltpu.sync_copy(dst_ref.at[pl.ds(base, C)], dst_v)
            pltpu.sync_copy(r_ref.at[pl.ds(base, C)], r_v)
            pltpu.sync_copy(norm_ref.at[pl.ds(base, C)], norm_v)
            for j in range(C // L):
                sl = pl.ds(j * L, L)
                gidx[sl] = r_v[sl] * n_nodes + src_v[sl]
            pltpu.async_copy(y_ref.at[gidx], rows, sem).wait()

            def scale(gi, carry2):
                nvec = norm_v[pl.ds(gi * L, L)]
                for lane in range(L):
                    nv = jnp.full((L,), nvec[lane], jnp.float32)
                    e = gi * L + lane
                    for j in range(H // L):
                        sl = pl.ds(j * L, L)
                        rows[e, sl] = rows[e, sl] * nv
                return carry2

            lax.fori_loop(0, C // L, scale, 0)
            pltpu.sync_copy(rows, sh_agg.at[dst_v], add=True)
            return carry

        lax.fori_loop(0, n_chunks, chunk, 0)
        plsc.subcore_barrier()

        pltpu.sync_copy(sh_agg.at[pl.ds(s * rpt, rpt)],
                        out_ref.at[c, pl.ds(s * rpt, rpt)])

    return body


def kernel(g, h, r, norm, embedding, W_rel, loop_weight, h_bias, ln_gamma,
           ln_beta):
    n_nodes, hdim = embedding.shape
    n_edges = g.shape[1]
    t_rel = W_rel.shape[0]
    assert hdim == H

    x = jnp.take(embedding, h, axis=0)
    src = g[0]
    dst = g[1]
    norm_flat = norm.reshape(-1)

    w_bd = jnp.zeros((t_rel, H, H), jnp.float32)
    for b in range(NB):
        w_bd = w_bd.at[:, b * BLK:(b + 1) * BLK, b * BLK:(b + 1) * BLK].set(
            W_rel[:, b])
    w_all = jnp.concatenate([w_bd, loop_weight[None]], axis=0)

    bm = 1000
    y = pl.pallas_call(
        _matmul_body,
        grid=(t_rel + 1, n_nodes // bm),
        in_specs=[
            pl.BlockSpec((bm, H), lambda t, i: (i, 0)),
            pl.BlockSpec((1, H, H), lambda t, i: (t, 0, 0)),
        ],
        out_specs=pl.BlockSpec((1, bm, H), lambda t, i: (t, i, 0)),
        out_shape=jax.ShapeDtypeStruct((t_rel + 1, n_nodes, H), jnp.float32),
    )(x, w_all)

    y_flat = y.reshape((t_rel + 1) * n_nodes, H)

    n_pad = ((n_nodes + NS * C - 1) // (NS * C)) * (NS * C)
    mesh = plsc.VectorSubcoreMesh(core_axis_name="c", subcore_axis_name="s",
                                  num_cores=NC, num_subcores=NS)
    partials = pl.kernel(
        _make_sc_body(n_nodes, n_pad, n_edges),
        out_type=jax.ShapeDtypeStruct((NC, n_pad, H), jnp.float32),
        mesh=mesh,
        scratch_types=[
            pltpu.VMEM((C,), jnp.int32),
            pltpu.VMEM((C,), jnp.int32),
            pltpu.VMEM((C,), jnp.int32),
            pltpu.VMEM((C,), jnp.float32),
            pltpu.VMEM((C,), jnp.int32),
            pltpu.VMEM((C, H), jnp.float32),
            pltpu.VMEM_SHARED((n_pad, H), jnp.float32),
            pltpu.SemaphoreType.DMA,
        ],
    )(y_flat, src, dst, r, norm_flat)

    params = jnp.stack([ln_gamma, ln_beta, h_bias])
    bm2 = 2000
    out = pl.pallas_call(
        _epilogue_body,
        grid=(n_nodes // bm2,),
        in_specs=[
            pl.BlockSpec((NC, bm2, H), lambda i: (0, i, 0)),
            pl.BlockSpec((bm2, H), lambda i: (i, 0)),
            pl.BlockSpec((3, H), lambda i: (0, 0)),
        ],
        out_specs=pl.BlockSpec((bm2, H), lambda i: (i, 0)),
        out_shape=jax.ShapeDtypeStruct((n_nodes, H), jnp.float32),
    )(partials, y[t_rel], params)

    return out
